# Initial kernel scaffold; baseline (speedup 1.0000x reference)
import jax
import jax.numpy as jnp
from jax.experimental import pallas as pl


def kernel(hidden_states, W, b, codevectors):
    raise NotImplementedError("write your pallas kernel here")


# speedup vs baseline: 4.8905x; 4.8905x over previous
import functools

import jax
import jax.numpy as jnp
from jax import lax
from jax.experimental import pallas as pl
from jax.experimental.pallas import tpu as pltpu
from jax.experimental.pallas import tpu_sc as plsc

_G = 2
_V = 320
_GV = _G * _V
_D = 128
_TOKENS = 2048
_TILE = 256
_NTILES = _TOKENS // _TILE


def _tc_body(x_ref, w_ref, b_ref, idx_ref, counts_ref, pexp_ref):
    i = pl.program_id(0)

    @pl.when(i == 0)
    def _init():
        counts_ref[...] = jnp.zeros_like(counts_ref)

    logits = lax.dot_general(
        x_ref[...], w_ref[...],
        dimension_numbers=(((1,), (1,)), ((), ())),
        preferred_element_type=jnp.float32,
    ) + b_ref[...]

    lane = lax.broadcasted_iota(jnp.int32, (_TILE, _GV), 1)
    g0 = lane < _V
    neg = jnp.float32(-jnp.inf)
    l0 = jnp.where(g0, logits, neg)
    l1 = jnp.where(g0, neg, logits)
    mx0 = jnp.max(l0, axis=1, keepdims=True)
    mx1 = jnp.max(l1, axis=1, keepdims=True)
    idx0 = jnp.min(jnp.where(l0 == mx0, lane, _GV), axis=1)
    idx1 = jnp.min(jnp.where(l1 == mx1, lane, 2 * _GV), axis=1)

    idx_ref[...] = jnp.concatenate([idx0[:, None], idx1[:, None]], axis=1)

    onehot = (lane == idx0[:, None]) | (lane == idx1[:, None])
    counts_ref[...] += jnp.sum(onehot.astype(jnp.float32), axis=0, keepdims=True)

    @pl.when(i == _NTILES - 1)
    def _finish():
        m = counts_ref[...] * jnp.float32(1.0 / _TOKENS)
        e = m * jnp.log(m + jnp.float32(1e-7))
        lane2 = lax.broadcasted_iota(jnp.int32, (1, _GV), 1)
        s0 = jnp.sum(jnp.where(lane2 < _V, e, 0.0))
        s1 = jnp.sum(jnp.where(lane2 >= _V, e, 0.0))
        pexp_ref[...] = (jnp.exp(-s0) + jnp.exp(-s1)).reshape(1, 1)


_tc_call = pl.pallas_call(
    _tc_body,
    grid=(_NTILES,),
    in_specs=[
        pl.BlockSpec((_TILE, 608), lambda i: (i, 0)),
        pl.BlockSpec((_GV, 608), lambda i: (0, 0)),
        pl.BlockSpec((1, _GV), lambda i: (0, 0)),
    ],
    out_specs=[
        pl.BlockSpec((_TILE, 2), lambda i: (i, 0)),
        pl.BlockSpec((1, _GV), lambda i: (0, 0)),
        pl.BlockSpec((1, 1), lambda i: (0, 0)),
    ],
    out_shape=[
        jax.ShapeDtypeStruct((_TOKENS, 2), jnp.int32),
        jax.ShapeDtypeStruct((1, _GV), jnp.float32),
        jax.ShapeDtypeStruct((1, 1), jnp.float32),
    ],
)

_NROWS = _TOKENS * _G
_NW = 32
_RPW = _NROWS // _NW


@functools.cache
def _make_sc_gather():
    mesh = plsc.VectorSubcoreMesh(core_axis_name="c", subcore_axis_name="s")

    @functools.partial(
        pl.kernel,
        mesh=mesh,
        out_type=jax.ShapeDtypeStruct((_NROWS, _D), jnp.float32),
        scratch_types=[
            pltpu.VMEM((_RPW,), jnp.int32),
            pltpu.VMEM((_RPW, _D), jnp.float32),
            pltpu.SemaphoreType.DMA,
        ],
    )
    def _sc_gather(idx_hbm, table_hbm, out_hbm, idx_v, rows_v, sem):
        wid = lax.axis_index("s") * 2 + lax.axis_index("c")
        base = wid * _RPW
        pltpu.sync_copy(idx_hbm.at[pl.ds(base, _RPW)], idx_v)
        pltpu.async_copy(table_hbm.at[idx_v], rows_v, sem).wait()
        pltpu.sync_copy(rows_v, out_hbm.at[pl.ds(base, _RPW)])

    return _sc_gather


def kernel(hidden_states, W, b, codevectors):
    B, S, H = hidden_states.shape
    x = hidden_states.reshape(B * S, H)
    idx_pairs, _, pexp = _tc_call(x, W, b.reshape(1, _GV))
    table = codevectors.reshape(_GV, _D)
    rows = _make_sc_gather()(idx_pairs.reshape(_NROWS), table)
    cv = rows.reshape(B, S, _G * _D)
    return cv, pexp[0, 0]
